# baseline (speedup 1.0000x reference)
import functools

import jax
import jax.numpy as jnp
import numpy as np
from jax import lax
from jax.experimental import pallas as pl
from jax.experimental.pallas import tpu as pltpu
from jax.experimental.pallas import tpu_sc as plsc

N = 10000
DEG = 16
DFEAT = 128
B = 1024
E1 = 64
E2 = 128

NC = 2
NS = 16
NW = NC * NS
NPAD = 10240

L1_PER_W = NPAD // NW
L1_CH = 16
L1_NCH = L1_PER_W // L1_CH
L2_PER_W = B // NW
L2_CH = 8
L2_NCH = L2_PER_W // L2_CH

_PK = plsc.PackFormat.INTERLEAVED


def _mesh():
    return plsc.VectorSubcoreMesh(core_axis_name="c", subcore_axis_name="s",
                                  num_cores=NC, num_subcores=NS)


_SC_PARAMS = pltpu.CompilerParams(use_tc_tiling_on_sc=False,
                                  needs_layout_passes=False)


def _wid():
    return lax.axis_index("s") * NC + lax.axis_index("c")


def _perm_idx(E):
    cols = np.arange(E).reshape(E // 32, 16, 2).transpose(0, 2, 1).reshape(-1)
    return cols


def _pack_weights(Wm, E):
    perm = jnp.asarray(_perm_idx(E))
    wa = Wm[:E].T[:, perm]
    wb = Wm[E:].T[:, perm]
    return jnp.concatenate([wa.reshape(-1), wb.reshape(-1)])


def _emb_body(f_ref, w_ref, b_ref, o_ref):
    x = jnp.dot(f_ref[...], w_ref[...], preferred_element_type=jnp.float32)
    o_ref[pl.ds(0, N), :] = jnp.maximum(x + b_ref[...], 0.0).astype(jnp.bfloat16)


def _emb(feat, W, b):
    return pl.pallas_call(
        _emb_body,
        out_shape=jax.ShapeDtypeStruct((NPAD, E1), jnp.bfloat16),
    )(feat, W, b)


def _agg_body(tbl_hbm, i1_hbm, i2_hbm, i3_hbm, self_hbm, w_hbm, out_hbm,
              idx1, idx2, idx3, selfall, spm, rA, rB, outA, outB, w_v,
              semA, semB, semOA, semOB,
              *, E, per_w, ch, nch, tbl_rows):
    wid = _wid()
    ng = E // 32
    wbase = wid * per_w
    cr = ch * DEG

    sid = lax.axis_index("s")
    trs = tbl_rows // NS
    pltpu.sync_copy(tbl_hbm.at[pl.ds(sid * trs, trs)],
                    spm.at[pl.ds(sid * trs, trs)])

    pltpu.sync_copy(w_hbm, w_v)
    pltpu.sync_copy(i1_hbm.at[pl.ds(wbase * DEG, per_w * DEG)], idx1)
    pltpu.sync_copy(i2_hbm.at[pl.ds(wbase * DEG, per_w * DEG)], idx2)
    pltpu.sync_copy(i3_hbm.at[pl.ds(wbase * DEG, per_w * DEG)], idx3)
    pltpu.sync_copy(self_hbm.at[pl.ds(wbase, per_w)], selfall)
    plsc.subcore_barrier()

    def wslice(kind, r, h, par):
        return w_v[pl.ds(kind * 3 * E + r * E + h * 32 + par * 16, 16)]

    wa = [[(wslice(0, r, h, 0), wslice(0, r, h, 1)) for h in range(ng)]
          for r in range(3)]
    wb = [[(wslice(1, r, h, 0), wslice(1, r, h, 1)) for h in range(ng)]
          for r in range(3)]

    def fire(c, rows, sem):
        for r, idx in enumerate((idx1, idx2, idx3)):
            pltpu.async_copy(
                spm.at[idx.at[pl.ds(c * cr, cr)]],
                rows.at[pl.ds(r * cr, cr)], sem)

    def wait_g(rows, sem):
        pltpu.make_async_copy(tbl_hbm.at[pl.ds(0, 3 * cr)], rows, sem).wait()

    def wait_o(outb, sem):
        pltpu.make_async_copy(outb, out_hbm.at[pl.ds(0, ch)], sem).wait()

    fire(0, rA, semA)
    fire(1, rB, semB)

    def compute(c, rows, outb):
        def node(i, carry):
            accA = [None] * ng
            accB = [None] * ng
            for r in range(3):
                rb = r * cr + i * DEG
                for h in range(ng):
                    sl = pl.ds(h * 32, 32)
                    se, so = plsc.unpack(rows[rb, sl], format=_PK)
                    for j in range(1, DEG):
                        xe, xo = plsc.unpack(rows[rb + j, sl], format=_PK)
                        se = se + xe
                        so = so + xo
                    me = se * (1.0 / DEG)
                    mo = so * (1.0 / DEG)
                    if r == 0:
                        accA[h] = [wa[r][h][0] * me, wa[r][h][1] * mo]
                        accB[h] = [wb[r][h][0] * me, wb[r][h][1] * mo]
                    else:
                        accA[h][0] = accA[h][0] + wa[r][h][0] * me
                        accA[h][1] = accA[h][1] + wa[r][h][1] * mo
                        accB[h][0] = accB[h][0] + wb[r][h][0] * me
                        accB[h][1] = accB[h][1] + wb[r][h][1] * mo
            for h in range(ng):
                sl = pl.ds(h * 32, 32)
                outb[i, sl] = plsc.pack(
                    accA[h][0], accA[h][1], format=_PK,
                    preferred_element_type=jnp.bfloat16)
                fe, fo = plsc.unpack(selfall[c * ch + i, sl], format=_PK)
                outb[i, pl.ds(E + h * 32, 32)] = plsc.pack(
                    fe - accB[h][0], fo - accB[h][1], format=_PK,
                    preferred_element_type=jnp.bfloat16)
            return carry

        lax.fori_loop(0, ch, node, 0)

    ni = nch // 2

    def iteration(i, carry):
        c0 = 2 * i
        c1 = 2 * i + 1
        wait_g(rA, semA)

        @pl.when(i > 0)
        def _():
            wait_o(outA, semOA)

        compute(c0, rA, outA)
        pltpu.async_copy(outA, out_hbm.at[pl.ds(wbase + c0 * ch, ch)], semOA)

        @pl.when(i < ni - 1)
        def _():
            fire(c0 + 2, rA, semA)

        wait_g(rB, semB)

        @pl.when(i > 0)
        def _():
            wait_o(outB, semOB)

        compute(c1, rB, outB)
        pltpu.async_copy(outB, out_hbm.at[pl.ds(wbase + c1 * ch, ch)], semOB)

        @pl.when(i < ni - 1)
        def _():
            fire(c1 + 2, rB, semB)

        return carry

    lax.fori_loop(0, ni, iteration, 0)
    wait_o(outA, semOA)
    wait_o(outB, semOB)


def _agg_kernel(tbl, i1f, i2f, i3f, selfv, wv, *, E, total, per_w, ch, nch):
    tbl_rows = tbl.shape[0]
    body = functools.partial(_agg_body, E=E, per_w=per_w, ch=ch, nch=nch,
                             tbl_rows=tbl_rows)
    f = functools.partial(
        pl.kernel,
        out_type=jax.ShapeDtypeStruct((total, 2 * E), jnp.bfloat16),
        mesh=_mesh(),
        compiler_params=_SC_PARAMS,
        scratch_types=[
            pltpu.VMEM((per_w * DEG,), jnp.int32),
            pltpu.VMEM((per_w * DEG,), jnp.int32),
            pltpu.VMEM((per_w * DEG,), jnp.int32),
            pltpu.VMEM((per_w, E), jnp.bfloat16),
            pltpu.VMEM_SHARED((tbl_rows, E), jnp.bfloat16),
            pltpu.VMEM((3 * ch * DEG, E), jnp.bfloat16),
            pltpu.VMEM((3 * ch * DEG, E), jnp.bfloat16),
            pltpu.VMEM((ch, 2 * E), jnp.bfloat16),
            pltpu.VMEM((ch, 2 * E), jnp.bfloat16),
            pltpu.VMEM((6 * E,), jnp.float32),
            pltpu.SemaphoreType.DMA,
            pltpu.SemaphoreType.DMA,
            pltpu.SemaphoreType.DMA,
            pltpu.SemaphoreType.DMA,
        ],
    )(body)
    return f(tbl, i1f, i2f, i3f, selfv, wv)


def _bg_body(nodes_hbm, a1_hbm, a2_hbm, a3_hbm, emb_hbm, int1_hbm,
             ab1_hbm, ab2_hbm, ab3_hbm, e0b_hbm, i1b_hbm,
             nd_v, ab1_v, ab2_v, ab3_v, e0b_v, i1b_v,
             s1, s2, s3, s4, s5):
    wid = _wid()
    base = wid * L2_PER_W
    pltpu.sync_copy(nodes_hbm.at[pl.ds(base, L2_PER_W)], nd_v)
    c1 = pltpu.async_copy(a1_hbm.at[nd_v], ab1_v, s1)
    c2 = pltpu.async_copy(a2_hbm.at[nd_v], ab2_v, s2)
    c3 = pltpu.async_copy(a3_hbm.at[nd_v], ab3_v, s3)
    c4 = pltpu.async_copy(emb_hbm.at[nd_v], e0b_v, s4)
    c5 = pltpu.async_copy(int1_hbm.at[nd_v], i1b_v, s5)
    c1.wait(); c2.wait(); c3.wait(); c4.wait(); c5.wait()
    pltpu.sync_copy(ab1_v, ab1_hbm.at[pl.ds(base, L2_PER_W)])
    pltpu.sync_copy(ab2_v, ab2_hbm.at[pl.ds(base, L2_PER_W)])
    pltpu.sync_copy(ab3_v, ab3_hbm.at[pl.ds(base, L2_PER_W)])
    pltpu.sync_copy(e0b_v, e0b_hbm.at[pl.ds(base, L2_PER_W)])
    pltpu.sync_copy(i1b_v, i1b_hbm.at[pl.ds(base, L2_PER_W)])


def _bgather(nodes, a1p, a2p, a3p, emb0, inter1):
    f = functools.partial(
        pl.kernel,
        out_type=[jax.ShapeDtypeStruct((B, DEG), jnp.int32),
                  jax.ShapeDtypeStruct((B, DEG), jnp.int32),
                  jax.ShapeDtypeStruct((B, DEG), jnp.int32),
                  jax.ShapeDtypeStruct((B, E1), jnp.bfloat16),
                  jax.ShapeDtypeStruct((B, E2), jnp.bfloat16)],
        mesh=_mesh(),
        compiler_params=_SC_PARAMS,
        scratch_types=[
            pltpu.VMEM((L2_PER_W,), jnp.int32),
            pltpu.VMEM((L2_PER_W, DEG), jnp.int32),
            pltpu.VMEM((L2_PER_W, DEG), jnp.int32),
            pltpu.VMEM((L2_PER_W, DEG), jnp.int32),
            pltpu.VMEM((L2_PER_W, E1), jnp.bfloat16),
            pltpu.VMEM((L2_PER_W, E2), jnp.bfloat16),
            pltpu.SemaphoreType.DMA,
            pltpu.SemaphoreType.DMA,
            pltpu.SemaphoreType.DMA,
            pltpu.SemaphoreType.DMA,
            pltpu.SemaphoreType.DMA,
        ],
    )(_bg_body)
    return f(nodes, a1p, a2p, a3p, emb0, inter1)


def _head_body(e_ref, i1_ref, i2_ref, w2a, w2b, w2c, b2_ref, lp_ref, w3_ref,
               b3_ref, o_ref):
    x = (jnp.dot(e_ref[...].astype(jnp.float32), w2a[...],
                 preferred_element_type=jnp.float32)
         + jnp.dot(i1_ref[...].astype(jnp.float32), w2b[...],
                   preferred_element_type=jnp.float32)
         + jnp.dot(i2_ref[...].astype(jnp.float32), w2c[...],
                   preferred_element_type=jnp.float32)
         + b2_ref[...])
    x = jnp.where(x >= 0.0, x, 0.3 * x)
    x = x + lp_ref[...]
    y = jnp.dot(x, w3_ref[...], preferred_element_type=jnp.float32) + b3_ref[...]
    o_ref[...] = jax.nn.sigmoid(y)


def _head(e0b, i1b, inter2, W2, b2, prior, W3, b3):
    return pl.pallas_call(
        _head_body,
        out_shape=jax.ShapeDtypeStruct((B, 1), jnp.float32),
    )(e0b, i1b, inter2, W2[:E1], W2[E1:E1 + E2], W2[E1 + E2:],
      b2.reshape(1, 2), jnp.log(prior).reshape(1, 2), W3, b3.reshape(1, 1))


def kernel(nodes, feat_data, adj1, adj2, adj3, prior, W_mlp, b_mlp,
           alpha1, alpha2, W2, b2, W3, b3):
    emb0 = _emb(feat_data, W_mlp, b_mlp.reshape(1, E1))

    w1 = _pack_weights(jax.nn.softmax(alpha1, axis=1), E1)
    w2v = _pack_weights(jax.nn.softmax(alpha2, axis=1), E2)

    pad = ((0, NPAD - N), (0, 0))
    a1p = jnp.pad(adj1, pad)
    a2p = jnp.pad(adj2, pad)
    a3p = jnp.pad(adj3, pad)

    inter1 = _agg_kernel(emb0, a1p.reshape(-1), a2p.reshape(-1),
                         a3p.reshape(-1), emb0, w1, E=E1, total=NPAD,
                         per_w=L1_PER_W, ch=L1_CH, nch=L1_NCH)

    ab1, ab2, ab3, e0b, i1b = _bgather(nodes, a1p, a2p, a3p, emb0, inter1)

    inter2 = _agg_kernel(inter1, ab1.reshape(-1), ab2.reshape(-1),
                         ab3.reshape(-1), i1b, w2v, E=E2, total=B,
                         per_w=L2_PER_W, ch=L2_CH, nch=L2_NCH)

    return _head(e0b, i1b, inter2, W2, b2, prior, W3, b3)
